# Initial kernel scaffold; baseline (speedup 1.0000x reference)
import jax
import jax.numpy as jnp
from jax.experimental import pallas as pl


def kernel(x, edge_index, batch, W1, b1, W2, b2, Wc, bc):
    raise NotImplementedError("write your pallas kernel here")


# speedup vs baseline: 30.6918x; 30.6918x over previous
import functools

import jax
import jax.numpy as jnp
from jax import lax
from jax.experimental import pallas as pl
from jax.experimental.pallas import tpu as pltpu
from jax.experimental.pallas import tpu_sc as plsc

N_NODES = 10000
N_EDGES = 320000
D_FEAT = 128
NUM_GRAPHS = 128
NUM_CLASSES = 10

NPAD = 10240
NC, NS = 2, 16
NW = NC * NS
WIN = 128
NWIN = 2528
WPT = NWIN // NW
EPAD = NWIN * WIN
RPT = NPAD // NS

_mesh = plsc.VectorSubcoreMesh(core_axis_name="c", subcore_axis_name="s")


def _deg_body(didx_hbm, ones_hbm, zeros_hbm, out_hbm, didx_v, ones_v, deg_sh):
    c = lax.axis_index("c")
    s = lax.axis_index("s")
    wid = c * NS + s
    pltpu.sync_copy(didx_hbm.at[wid], didx_v)
    pltpu.sync_copy(ones_hbm, ones_v)

    @pl.when(s == 0)
    def _():
        pltpu.sync_copy(zeros_hbm, deg_sh)

    plsc.subcore_barrier()

    def win(j, carry):
        pltpu.sync_copy(ones_v, deg_sh.at[didx_v.at[j]], add=True)
        return carry

    lax.fori_loop(0, WPT, win, 0)
    plsc.subcore_barrier()
    pltpu.sync_copy(deg_sh.at[pl.ds(s * RPT, RPT)],
                    out_hbm.at[c, pl.ds(s * RPT, RPT)])


_deg_call = pl.kernel(
    _deg_body,
    out_type=jax.ShapeDtypeStruct((NC, NPAD), jnp.float32),
    mesh=_mesh,
    scratch_types=[
        pltpu.VMEM((WPT, WIN), jnp.int32),
        pltpu.VMEM((WIN,), jnp.float32),
        pltpu.VMEM_SHARED((NPAD,), jnp.float32),
    ],
)


def _agg_body(table_hbm, sidx_hbm, didx_hbm, zeros_hbm, out_hbm,
              sidx_v, didx_v, rows_v, acc_sh, sem):
    c = lax.axis_index("c")
    s = lax.axis_index("s")
    wid = c * NS + s
    pltpu.sync_copy(sidx_hbm.at[wid], sidx_v)
    pltpu.sync_copy(didx_hbm.at[wid], didx_v)

    @pl.when(s == 0)
    def _():
        pltpu.sync_copy(zeros_hbm, acc_sh)

    plsc.subcore_barrier()

    def win(j, carry):
        pltpu.async_copy(table_hbm.at[sidx_v.at[j]], rows_v, sem).wait()
        pltpu.sync_copy(rows_v, acc_sh.at[didx_v.at[j]], add=True)
        return carry

    lax.fori_loop(0, WPT, win, 0)
    plsc.subcore_barrier()
    pltpu.sync_copy(acc_sh.at[pl.ds(s * RPT, RPT)],
                    out_hbm.at[c, pl.ds(s * RPT, RPT)])


def _make_agg(d):
    return pl.kernel(
        _agg_body,
        out_type=jax.ShapeDtypeStruct((NC, NPAD, d), jnp.float32),
        mesh=_mesh,
        compiler_params=pltpu.CompilerParams(use_tc_tiling_on_sc=False),
        scratch_types=[
            pltpu.VMEM((WPT, WIN), jnp.int32),
            pltpu.VMEM((WPT, WIN), jnp.int32),
            pltpu.VMEM((WIN, d), jnp.float32),
            pltpu.VMEM_SHARED((NPAD, d), jnp.float32),
            pltpu.SemaphoreType.DMA,
        ],
    )


_agg64 = _make_agg(64)
_agg32 = _make_agg(32)


def _tc1_body(x_ref, w1_ref, degp_ref, mask_ref, hs1_ref, dinv_ref):
    deg = degp_ref[0] + degp_ref[1] + 1.0
    dinv = lax.rsqrt(deg) * mask_ref[...]
    h = jnp.dot(x_ref[...], w1_ref[...], preferred_element_type=jnp.float32)
    hs1_ref[...] = h * dinv
    dinv_ref[...] = dinv


_tc1 = pl.pallas_call(
    _tc1_body,
    out_shape=(jax.ShapeDtypeStruct((NPAD, 64), jnp.float32),
               jax.ShapeDtypeStruct((NPAD, 1), jnp.float32)),
)


def _tc2_body(acc_ref, hs1_ref, dinv_ref, b1_ref, w2_ref, hs2_ref):
    t = acc_ref[0] + acc_ref[1] + hs1_ref[...]
    z1 = jnp.maximum(t * dinv_ref[...] + b1_ref[...], 0.0)
    hs2_ref[...] = jnp.dot(z1, w2_ref[...],
                           preferred_element_type=jnp.float32) * dinv_ref[...]


_tc2 = pl.pallas_call(
    _tc2_body,
    out_shape=jax.ShapeDtypeStruct((NPAD, 32), jnp.float32),
)


def _tc3_body(acc_ref, hs2_ref, dinv_ref, b2_ref, batch_ref, wc_ref, bc_ref,
              out_ref):
    z2 = jnp.maximum(
        (acc_ref[0] + acc_ref[1] + hs2_ref[...]) * dinv_ref[...] + b2_ref[...],
        0.0)
    gids = lax.broadcasted_iota(jnp.int32, (NUM_GRAPHS, NPAD), 0)
    onehot = jnp.where(gids == batch_ref[...], 1.0, 0.0)
    sums = jnp.dot(onehot, z2, preferred_element_type=jnp.float32)
    counts = jnp.sum(onehot, axis=1, keepdims=True)
    pooled = sums / jnp.maximum(counts, 1.0)
    logits = jnp.dot(pooled, wc_ref[...],
                     preferred_element_type=jnp.float32) + bc_ref[...]
    m = jnp.max(logits, axis=1, keepdims=True)
    lse = jnp.log(jnp.sum(jnp.exp(logits - m), axis=1, keepdims=True)) + m
    out_ref[...] = logits - lse


_tc3 = pl.pallas_call(
    _tc3_body,
    out_shape=jax.ShapeDtypeStruct((NUM_GRAPHS, NUM_CLASSES), jnp.float32),
)


@jax.jit
def kernel(x, edge_index, batch, W1, b1, W2, b2, Wc, bc):
    f32 = jnp.float32
    padn = EPAD - N_EDGES
    padrows = (N_NODES
               + jnp.arange(padn, dtype=jnp.int32) % (NPAD - N_NODES))
    src_p = jnp.concatenate([edge_index[0], padrows]).reshape(NW, WPT, WIN)
    dst_p = jnp.concatenate([edge_index[1], padrows]).reshape(NW, WPT, WIN)

    x_p = jnp.pad(x, ((0, NPAD - N_NODES), (0, 0)))
    batch_p = jnp.pad(batch, (0, NPAD - N_NODES),
                      constant_values=-1).reshape(1, NPAD)
    mask = (jnp.arange(NPAD) < N_NODES).astype(f32).reshape(NPAD, 1)
    zeros_nd = jnp.zeros((NPAD, 64), f32)
    zeros_1 = jnp.zeros((NPAD,), f32)
    ones_w = jnp.ones((WIN,), f32)

    degp = _deg_call(dst_p, ones_w, zeros_1)
    hs1, dinv = _tc1(x_p, W1, degp.reshape(NC, NPAD, 1), mask)
    acc1 = _agg64(hs1, src_p, dst_p, zeros_nd)
    hs2 = _tc2(acc1, hs1, dinv, b1.reshape(1, 64), W2)
    acc2 = _agg32(hs2, src_p, dst_p, zeros_nd[:, :32])
    return _tc3(acc2, hs2, dinv, b2.reshape(1, 32), batch_p, Wc,
                bc.reshape(1, NUM_CLASSES))
